# baseline (speedup 1.0000x reference)
import functools

import jax
import jax.numpy as jnp
from jax import lax
from jax.experimental import pallas as pl
from jax.experimental.pallas import tpu as pltpu
from jax.experimental.pallas import tpu_sc as plsc

_B, _T, _N, _DIN, _H = 16, 12, 2048, 3, 32
_BT = _B * _T
_C = 4 * _H
_LANES = 16
_RPW = 8


def _sc_emb_rows(idx2, tab):
    info = plsc.get_sparse_core_info()
    nc = info.num_cores
    n_active = _BT // _RPW

    def body(idx2_hbm, tab_hbm, out_hbm, idxd_v, idxw_v, day_v, week_v,
             comb_v, sem_d, sem_w):
        wid = lax.axis_index("s") * nc + lax.axis_index("c")

        @pl.when(wid < n_active)
        def _():
            base = wid * _RPW
            pltpu.sync_copy(idx2_hbm.at[pl.ds(base, _RPW)], idxd_v)
            pltpu.sync_copy(idx2_hbm.at[pl.ds(_BT + base, _RPW)], idxw_v)
            pltpu.async_copy(tab_hbm.at[idxd_v], day_v, sem_d).wait()
            pltpu.async_copy(tab_hbm.at[idxw_v], week_v, sem_w).wait()
            for r in range(_RPW):
                for c in range(_C // _LANES):
                    sl = pl.ds(c * _LANES, _LANES)
                    comb_v[r, sl] = day_v[r, sl] + week_v[r, sl]
            pltpu.sync_copy(comb_v, out_hbm.at[pl.ds(base, _RPW)])

    fn = functools.partial(
        pl.kernel,
        mesh=plsc.VectorSubcoreMesh(core_axis_name="c", subcore_axis_name="s"),
        out_type=jax.ShapeDtypeStruct((_BT, _C), jnp.float32),
        scratch_types=[
            pltpu.VMEM((_RPW,), jnp.int32),
            pltpu.VMEM((_RPW,), jnp.int32),
            pltpu.VMEM((_RPW, _C), jnp.float32),
            pltpu.VMEM((_RPW, _C), jnp.float32),
            pltpu.VMEM((_RPW, _C), jnp.float32),
            pltpu.SemaphoreType.DMA,
            pltpu.SemaphoreType.DMA,
        ],
    )(body)
    return fn(idx2, tab)


_G = 16


def _tc_body(x_ref, w_ref, b_ref, bc_ref, o_ref):
    for g in range(_G):
        xt = x_ref[:, g]
        h = lax.dot_general(xt, w_ref[...], (((0,), (0,)), ((), ())),
                            preferred_element_type=jnp.float32)
        v = h + b_ref[g] + bc_ref[0]
        lane = lax.broadcasted_iota(jnp.int32, v.shape, 1)
        o_ref[g] = jnp.where(lane >= 3 * _H, jnp.maximum(v, 0.0), v)


def _tc_fused(xt, w_cat, bias3, b_cat):
    return pl.pallas_call(
        _tc_body,
        grid=(_BT // _G,),
        in_specs=[
            pl.BlockSpec((_DIN, _G, _N), lambda i: (0, i, 0)),
            pl.BlockSpec((_DIN, _C), lambda i: (0, 0)),
            pl.BlockSpec((_G, 1, _C), lambda i: (i, 0, 0)),
            pl.BlockSpec((1, _C), lambda i: (0, 0)),
        ],
        out_specs=pl.BlockSpec((_G, _N, _C), lambda i: (i, 0, 0)),
        out_shape=jax.ShapeDtypeStruct((_BT, _N, _C), jnp.float32),
    )(xt, w_cat, bias3, b_cat)


def kernel(x, t_day, t_week, W_tok, b_tok, day_table, week_table, W_sp, b_sp):
    idx2 = jnp.concatenate([t_day.reshape(-1).astype(jnp.int32),
                            t_week.reshape(-1).astype(jnp.int32) + 288])
    tab = (jnp.zeros((296, _C), jnp.float32)
           .at[:288, _H : 2 * _H].set(day_table)
           .at[288:295, 2 * _H : 3 * _H].set(week_table))
    emb = _sc_emb_rows(idx2, tab)
    bias3 = emb.reshape(_BT, 1, _C)
    w_cat = (jnp.zeros((_DIN, _C), jnp.float32)
             .at[:, : _H].set(W_tok.T)
             .at[:, 3 * _H :].set(W_sp.T))
    b_cat = jnp.concatenate(
        [b_tok, jnp.zeros((2 * _H,), jnp.float32), b_sp]).reshape(1, _C)
    xt = jnp.transpose(x.reshape(_BT, _N, _DIN), (2, 0, 1))
    out = _tc_fused(xt, w_cat, bias3, b_cat)
    return out.reshape(_B, _T, _N, _C)


# speedup vs baseline: 1.0494x; 1.0494x over previous
import functools

import jax
import jax.numpy as jnp
from jax import lax
from jax.experimental import pallas as pl
from jax.experimental.pallas import tpu as pltpu
from jax.experimental.pallas import tpu_sc as plsc

_B, _T, _N, _DIN, _H = 16, 12, 2048, 3, 32
_BT = _B * _T
_C = 4 * _H
_LANES = 16
_RPW = 8


def _sc_emb_rows(idx2, tab):
    info = plsc.get_sparse_core_info()
    nc = info.num_cores
    n_active = _BT // _RPW

    def body(idx2_hbm, tab_hbm, out_hbm, idxd_v, idxw_v, day_v, week_v,
             comb_v, sem_d, sem_w):
        wid = lax.axis_index("s") * nc + lax.axis_index("c")

        @pl.when(wid < n_active)
        def _():
            base = wid * _RPW
            pltpu.sync_copy(idx2_hbm.at[pl.ds(base, _RPW)], idxd_v)
            pltpu.sync_copy(idx2_hbm.at[pl.ds(_BT + base, _RPW)], idxw_v)
            pltpu.async_copy(tab_hbm.at[idxd_v], day_v, sem_d).wait()
            pltpu.async_copy(tab_hbm.at[idxw_v], week_v, sem_w).wait()
            for r in range(_RPW):
                for c in range(_C // _LANES):
                    sl = pl.ds(c * _LANES, _LANES)
                    comb_v[r, sl] = day_v[r, sl] + week_v[r, sl]
            pltpu.sync_copy(comb_v, out_hbm.at[pl.ds(base, _RPW)])

    fn = functools.partial(
        pl.kernel,
        mesh=plsc.VectorSubcoreMesh(core_axis_name="c", subcore_axis_name="s"),
        out_type=jax.ShapeDtypeStruct((_BT, _C), jnp.float32),
        scratch_types=[
            pltpu.VMEM((_RPW,), jnp.int32),
            pltpu.VMEM((_RPW,), jnp.int32),
            pltpu.VMEM((_RPW, _C), jnp.float32),
            pltpu.VMEM((_RPW, _C), jnp.float32),
            pltpu.VMEM((_RPW, _C), jnp.float32),
            pltpu.SemaphoreType.DMA,
            pltpu.SemaphoreType.DMA,
        ],
    )(body)
    return fn(idx2, tab)


_G = 16


def _tc_body(x_ref, w_ref, b_ref, bc_ref, o_ref):
    for g in range(_G):
        xt = x_ref[g]
        h = lax.dot_general(xt, w_ref[...], (((0,), (0,)), ((), ())),
                            preferred_element_type=jnp.float32)
        v = h + b_ref[g] + bc_ref[0]
        lane = lax.broadcasted_iota(jnp.int32, v.shape, 1)
        o_ref[g] = jnp.where(lane >= 3 * _H, jnp.maximum(v, 0.0), v)


def _tc_fused(xt, w_cat, bias3, b_cat):
    return pl.pallas_call(
        _tc_body,
        grid=(_BT // _G,),
        in_specs=[
            pl.BlockSpec((_G, _DIN, _N), lambda i: (i, 0, 0)),
            pl.BlockSpec((_DIN, _C), lambda i: (0, 0)),
            pl.BlockSpec((_G, 1, _C), lambda i: (i, 0, 0)),
            pl.BlockSpec((1, _C), lambda i: (0, 0)),
        ],
        out_specs=pl.BlockSpec((_G, _N, _C), lambda i: (i, 0, 0)),
        out_shape=jax.ShapeDtypeStruct((_BT, _N, _C), jnp.float32),
    )(xt, w_cat, bias3, b_cat)


def kernel(x, t_day, t_week, W_tok, b_tok, day_table, week_table, W_sp, b_sp):
    idx2 = jnp.concatenate([t_day.reshape(-1).astype(jnp.int32),
                            t_week.reshape(-1).astype(jnp.int32) + 288])
    tab = (jnp.zeros((296, _C), jnp.float32)
           .at[:288, _H : 2 * _H].set(day_table)
           .at[288:295, 2 * _H : 3 * _H].set(week_table))
    emb = _sc_emb_rows(idx2, tab)
    bias3 = emb.reshape(_BT, 1, _C)
    w_cat = (jnp.zeros((_DIN, _C), jnp.float32)
             .at[:, : _H].set(W_tok.T)
             .at[:, 3 * _H :].set(W_sp.T))
    b_cat = jnp.concatenate(
        [b_tok, jnp.zeros((2 * _H,), jnp.float32), b_sp]).reshape(1, _C)
    xt = jnp.swapaxes(x.reshape(_BT, _N, _DIN), 1, 2)
    out = _tc_fused(xt, w_cat, bias3, b_cat)
    return out.reshape(_B, _T, _N, _C)
